# baseline (speedup 1.0000x reference)
import jax
import jax.numpy as jnp
from jax import lax
from jax.experimental import pallas as pl
from jax.experimental.pallas import tpu as pltpu
from jax.experimental.pallas import tpu_sc as plsc

B = 16384
D = 32
NC = 2
NS = 16
L = 16
NW = NC * NS
BPW = B // NW


def _body(uidx_hbm, iidx_hbm, utab_hbm, itab_hbm, out_hbm,
          idx_u, idx_i, rows_u, rows_i, out_v, sem_u, sem_i):
    wid = lax.axis_index("s") * NC + lax.axis_index("c")
    base = wid * BPW

    pltpu.sync_copy(uidx_hbm.at[pl.ds(base, BPW)], idx_u)
    pltpu.sync_copy(iidx_hbm.at[pl.ds(base, BPW)], idx_i)

    cu = pltpu.async_copy(utab_hbm.at[idx_u], rows_u, sem_u)
    ci = pltpu.async_copy(itab_hbm.at[idx_i], rows_i, sem_i)
    cu.wait()
    ci.wait()

    lanes = lax.iota(jnp.int32, L)

    def chunk_body(c, _):
        rows = c * L + lanes

        def d_body(d, acc):
            col = jnp.zeros((L,), jnp.int32) + d
            u = plsc.load_gather(rows_u, [rows, col])
            v = plsc.load_gather(rows_i, [rows, col])
            return acc + u * v

        acc = lax.fori_loop(0, D, d_body, jnp.zeros((L,), jnp.float32))
        out_v[pl.ds(c * L, L)] = acc
        return 0

    lax.fori_loop(0, BPW // L, chunk_body, 0)

    pltpu.sync_copy(out_v, out_hbm.at[pl.ds(base, BPW)])


@jax.jit
def _mf_rating(user_indices, item_indices, user_table, item_table):
    mesh = plsc.VectorSubcoreMesh(
        core_axis_name="c", subcore_axis_name="s",
        num_cores=NC, num_subcores=NS)
    return pl.kernel(
        _body,
        out_type=jax.ShapeDtypeStruct((B,), jnp.float32),
        mesh=mesh,
        compiler_params=pltpu.CompilerParams(
            needs_layout_passes=False, use_tc_tiling_on_sc=False),
        scratch_types=[
            pltpu.VMEM((BPW,), jnp.int32),
            pltpu.VMEM((BPW,), jnp.int32),
            pltpu.VMEM((BPW, D), jnp.float32),
            pltpu.VMEM((BPW, D), jnp.float32),
            pltpu.VMEM((BPW,), jnp.float32),
            pltpu.SemaphoreType.DMA,
            pltpu.SemaphoreType.DMA,
        ],
    )(user_indices, item_indices, user_table, item_table)


def kernel(user_indices, item_indices, user_table, item_table):
    return _mf_rating(user_indices, item_indices, user_table, item_table)


# speedup vs baseline: 4.1599x; 4.1599x over previous
import jax
import jax.numpy as jnp
from jax import lax
from jax.experimental import pallas as pl
from jax.experimental.pallas import tpu as pltpu
from jax.experimental.pallas import tpu_sc as plsc

B = 16384
D = 32
V = 1000000
NC = 2
NS = 16
L = 16
NW = NC * NS
BPW = B // NW
SUB = 4
NSUB = L // SUB
NGRP = BPW // L


def _body(uidx_hbm, iidx_hbm, utab_hbm, itab_hbm, out_hbm,
          uidx_v, iidx_v, ublk, iblk, obuf, sem):
    wid = lax.axis_index("s") * NC + lax.axis_index("c")
    base = wid * BPW

    pltpu.sync_copy(uidx_hbm.at[pl.ds(base, BPW)], uidx_v)
    pltpu.sync_copy(iidx_hbm.at[pl.ds(base, BPW)], iidx_v)

    lanes = lax.iota(jnp.int32, L)
    d_lo = lax.iota(jnp.int32, L)
    d_hi = d_lo + L

    def fire(sub, uvec, ivec, b):
        cps = []
        for k in range(SUB):
            jj = sub * SUB + k
            ru = uvec[jj]
            ri = ivec[jj]
            tcu = pl.multiple_of(
                lax.shift_right_logical(ru, 7) * jnp.int32(128), 128)
            tci = pl.multiple_of(
                lax.shift_right_logical(ri, 7) * jnp.int32(128), 128)
            cps.append(pltpu.async_copy(
                utab_hbm.at[:, pl.ds(tcu, 128)], ublk.at[b, k], sem))
            cps.append(pltpu.async_copy(
                itab_hbm.at[:, pl.ds(tci, 128)], iblk.at[b, k], sem))
        return cps

    def compute(sub, uvec, ivec, b, res):
        for k in range(SUB):
            jj = sub * SUB + k
            ucol = jnp.zeros((L,), jnp.int32) + lax.bitwise_and(
                uvec[jj], jnp.int32(127))
            icol = jnp.zeros((L,), jnp.int32) + lax.bitwise_and(
                ivec[jj], jnp.int32(127))
            ub = ublk.at[b, k]
            ib = iblk.at[b, k]
            u0 = plsc.load_gather(ub, [d_lo, ucol])
            u1 = plsc.load_gather(ub, [d_hi, ucol])
            v0 = plsc.load_gather(ib, [d_lo, icol])
            v1 = plsc.load_gather(ib, [d_hi, icol])
            p = u0 * v0 + u1 * v1
            s = lax.reduce_sum_p.bind(p, axes=(0,))
            res = jnp.where(lanes == jj, s, res)
        return res

    def group(g, _):
        uvec = uidx_v[pl.ds(g * L, L)]
        ivec = iidx_v[pl.ds(g * L, L)]
        res = jnp.zeros((L,), jnp.float32)
        cps = fire(0, uvec, ivec, 0)
        for sub in range(NSUB):
            nxt = []
            if sub + 1 < NSUB:
                nxt = fire(sub + 1, uvec, ivec, (sub + 1) % 2)
            for cp in cps:
                cp.wait()
            res = compute(sub, uvec, ivec, sub % 2, res)
            cps = nxt
        obuf[pl.ds(g * L, L)] = res
        return 0

    lax.fori_loop(0, NGRP, group, 0)

    pltpu.sync_copy(obuf, out_hbm.at[pl.ds(base, BPW)])


@jax.jit
def _mf_rating(user_indices, item_indices, user_table, item_table):
    mesh = plsc.VectorSubcoreMesh(
        core_axis_name="c", subcore_axis_name="s",
        num_cores=NC, num_subcores=NS)
    return pl.kernel(
        _body,
        out_type=jax.ShapeDtypeStruct((B,), jnp.float32),
        mesh=mesh,
        compiler_params=pltpu.CompilerParams(needs_layout_passes=False),
        scratch_types=[
            pltpu.VMEM((BPW,), jnp.int32),
            pltpu.VMEM((BPW,), jnp.int32),
            pltpu.VMEM((2, SUB, D, 128), jnp.float32),
            pltpu.VMEM((2, SUB, D, 128), jnp.float32),
            pltpu.VMEM((BPW,), jnp.float32),
            pltpu.SemaphoreType.DMA,
        ],
    )(user_indices, item_indices, user_table.T, item_table.T)


def kernel(user_indices, item_indices, user_table, item_table):
    return _mf_rating(user_indices, item_indices, user_table, item_table)
